# baseline (speedup 1.0000x reference)
import functools

import jax
import jax.numpy as jnp
from jax import lax
from jax.experimental import pallas as pl
from jax.experimental.pallas import tpu as pltpu
from jax.experimental.pallas import tpu_sc as plsc

_SHAPE = (4, 8)
_DIM = 32
_NUM_WORKERS = 32
_NC = 2


@functools.partial(jax.jit, static_argnames=("b_per_w",))
def _gather_sc(table, index, *, b_per_w):
    mesh = plsc.VectorSubcoreMesh(core_axis_name="c", subcore_axis_name="s")

    @functools.partial(
        pl.kernel,
        mesh=mesh,
        out_type=jax.ShapeDtypeStruct((index.shape[0], _DIM), jnp.float32),
        scratch_types=[
            pltpu.VMEM((b_per_w,), jnp.int32),
            pltpu.VMEM((b_per_w, _DIM), jnp.float32),
            pltpu.SemaphoreType.DMA,
        ],
        compiler_params=pltpu.CompilerParams(use_tc_tiling_on_sc=False),
    )
    def k(table_hbm, idx_hbm, out_hbm, idx_v, rows_v, sem):
        wid = lax.axis_index("s") * _NC + lax.axis_index("c")
        base = wid * b_per_w
        pltpu.sync_copy(idx_hbm.at[pl.ds(base, b_per_w)], idx_v)
        pltpu.async_copy(table_hbm.at[idx_v], rows_v, sem).wait()
        pltpu.sync_copy(rows_v, out_hbm.at[pl.ds(base, b_per_w)])

    return k(table, index)


def kernel(index, table):
    batch = index.shape[0]
    b_per_w = batch // _NUM_WORKERS
    out = _gather_sc(table, index.astype(jnp.int32), b_per_w=b_per_w)
    return out.reshape((batch,) + _SHAPE)


# speedup vs baseline: 3.8534x; 3.8534x over previous
import functools

import jax
import jax.numpy as jnp
from jax import lax
from jax.experimental import pallas as pl
from jax.experimental.pallas import tpu as pltpu
from jax.experimental.pallas import tpu_sc as plsc

_SHAPE = (4, 8)
_DIM = 32
_NUM_WORKERS = 32
_NC = 2
_ROWS = 1000000
_W = 128


@jax.jit
def _gather_sc(tableT3, index):
    batch = index.shape[0]
    b_per_w = batch // _NUM_WORKERS
    n_chunks = b_per_w // 16
    mesh = plsc.VectorSubcoreMesh(core_axis_name="c", subcore_axis_name="s")

    @functools.partial(
        pl.kernel,
        mesh=mesh,
        out_type=jax.ShapeDtypeStruct((4, 8, batch), jnp.float32),
        scratch_types=[
            pltpu.VMEM((b_per_w,), jnp.int32),
            pltpu.VMEM((4, 8, 16 * _W), jnp.float32),
            pltpu.VMEM((4, 8, b_per_w), jnp.float32),
            pltpu.SemaphoreType.DMA,
        ],
        compiler_params=pltpu.CompilerParams(needs_layout_passes=False),
    )
    def k(tableT3_hbm, idx_hbm, outT3_hbm, idx_v, stage_v, out_v, sem):
        wid = lax.axis_index("s") * _NC + lax.axis_index("c")
        base = wid * b_per_w
        pltpu.sync_copy(idx_hbm.at[pl.ds(base, b_per_w)], idx_v)

        lanes = lax.broadcasted_iota(jnp.int32, (16,), 0)

        def chunk_copies(j):
            v = idx_v[pl.ds(j * 16, 16)]
            copies = []
            for kk in range(16):
                rt = pl.multiple_of((v[kk] >> 7) << 7, _W)
                copies.append(pltpu.make_async_copy(
                    tableT3_hbm.at[:, :, pl.ds(rt, _W)],
                    stage_v.at[:, :, pl.ds(kk * _W, _W)],
                    sem,
                ))
            return copies

        def select(j):
            v = idx_v[pl.ds(j * 16, 16)]
            off = lanes * _W + (v & (_W - 1))
            for c8 in range(4):
                c8v = jnp.full((16,), c8, jnp.int32)
                for cm in range(8):
                    cmv = jnp.full((16,), cm, jnp.int32)
                    vals = plsc.load_gather(stage_v, [c8v, cmv, off])
                    out_v[c8, cm, pl.ds(j * 16, 16)] = vals

        @pl.loop(0, n_chunks)
        def _run(j):
            for c in chunk_copies(j):
                c.start()
            for c in chunk_copies(j):
                c.wait()
            select(j)

        pltpu.sync_copy(out_v, outT3_hbm.at[:, :, pl.ds(base, b_per_w)])

    return k(tableT3, index)


def kernel(index, table):
    batch = index.shape[0]
    tableT3 = table.T.reshape(4, 8, _ROWS)
    outT3 = _gather_sc(tableT3, index.astype(jnp.int32))
    return outT3.reshape(_DIM, batch).T.reshape((batch,) + _SHAPE)
